# baseline (speedup 1.0000x reference)
import jax
import jax.numpy as jnp
from jax.experimental import pallas as pl

BB = 2


def _argmax_onehot_kernel(x_ref, o_ref):
    xb = x_ref[...]
    _, R, W = xb.shape
    half = W // 2
    m = jnp.max(xb, axis=1, keepdims=True)
    r = jnp.argmax(xb, axis=1).astype(jnp.int32)[:, None, :]
    lane = jax.lax.broadcasted_iota(jnp.int32, m.shape, 2)
    k = 2 * r + (lane >= half).astype(jnp.int32)
    m_sw = jnp.roll(m, half, axis=2)
    k_sw = jnp.roll(k, half, axis=2)
    win = (m > m_sw) | ((m == m_sw) & (k < k_sw))
    t = jnp.where(win, r, -1)
    row_iota = jax.lax.broadcasted_iota(jnp.int32, xb.shape, 1)
    o_ref[...] = (row_iota == t).astype(jnp.float32)


def kernel(x):
    B, Kdim, L = x.shape
    xv = x.reshape(B, Kdim // 2, 2 * L)
    out = pl.pallas_call(
        _argmax_onehot_kernel,
        grid=(B // BB,),
        in_specs=[pl.BlockSpec((BB, Kdim // 2, 2 * L), lambda b: (b, 0, 0))],
        out_specs=pl.BlockSpec((BB, Kdim // 2, 2 * L), lambda b: (b, 0, 0)),
        out_shape=jax.ShapeDtypeStruct((B, Kdim // 2, 2 * L), x.dtype),
    )(xv)
    return out.reshape(B, Kdim, L)


# speedup vs baseline: 1.7202x; 1.7202x over previous
import jax
import jax.numpy as jnp
from jax.experimental import pallas as pl

BB = 4


def _stargmax_kernel(x_ref, o_ref):
    xb = x_ref[...]
    K = xb.shape[1]
    mx = jnp.max(xb, axis=1, keepdims=True)
    iota = jax.lax.broadcasted_iota(jnp.int32, xb.shape, 1)
    cand = jnp.where(xb == mx, iota, K)
    am = jnp.min(cand, axis=1, keepdims=True)
    o_ref[...] = (iota == am).astype(jnp.float32)


def kernel(x):
    B, Kdim, L = x.shape
    grid = (B // BB,)
    return pl.pallas_call(
        _stargmax_kernel,
        grid=grid,
        in_specs=[pl.BlockSpec((BB, Kdim, L), lambda b: (b, 0, 0))],
        out_specs=pl.BlockSpec((BB, Kdim, L), lambda b: (b, 0, 0)),
        out_shape=jax.ShapeDtypeStruct((B, Kdim, L), x.dtype),
    )(x)
